# baseline (speedup 1.0000x reference)
import functools

import jax
import jax.numpy as jnp
from jax import lax
from jax.experimental import pallas as pl
from jax.experimental.pallas import tpu as pltpu

B, S, D, H, E, K = 2, 2048, 1024, 1024, 8, 2
T = B * S
TM = 512
NEG = -3.0e38


def _moe_dense_body(x_ref, mask_ref, wg_ref, w1_ref, b1_ref, w2_ref, b2_ref,
                    out_ref, acc_ref, meta_ref, w1b_ref, w2b_ref):
    e = pl.program_id(0)
    t = pl.program_id(1)
    x = x_ref[...]
    rows = pl.ds(t * TM, TM)

    @pl.when(t == 0)
    def _cast_weights():
        w1b_ref[...] = w1_ref[0].astype(jnp.bfloat16)
        w2b_ref[...] = w2_ref[0].astype(jnp.bfloat16)

    @pl.when(e == 0)
    def _gating():
        maskf = mask_ref[...].astype(jnp.float32)
        logits = jnp.dot(x, wg_ref[...], preferred_element_type=jnp.float32)
        col = lax.broadcasted_iota(jnp.int32, logits.shape, 1)
        logits = jnp.where(col < E, logits, NEG)
        m1 = jnp.max(logits, axis=1, keepdims=True)
        i1 = jnp.min(jnp.where(logits == m1, col, E), axis=1, keepdims=True)
        l2 = jnp.where(col == i1, NEG, logits)
        m2 = jnp.max(l2, axis=1, keepdims=True)
        i2 = jnp.min(jnp.where(l2 == m2, col, E), axis=1, keepdims=True)
        d = jnp.exp(m2 - m1)
        g1 = (1.0 / (1.0 + d)) * maskf
        g2 = (d / (1.0 + d)) * maskf
        meta_ref[0, rows, :] = i1.astype(jnp.float32)
        meta_ref[1, rows, :] = g1
        meta_ref[2, rows, :] = i2.astype(jnp.float32)
        meta_ref[3, rows, :] = g2

    ef = e.astype(jnp.float32)
    ge = (jnp.where(meta_ref[0, rows, :] == ef, meta_ref[1, rows, :], 0.0)
          + jnp.where(meta_ref[2, rows, :] == ef, meta_ref[3, rows, :], 0.0))
    xb = x.astype(jnp.bfloat16)
    h = jnp.maximum(
        jnp.dot(xb, w1b_ref[...], preferred_element_type=jnp.float32)
        + b1_ref[0], 0.0)
    oe = jnp.dot(h.astype(jnp.bfloat16), w2b_ref[...],
                 preferred_element_type=jnp.float32) + b2_ref[0]
    contrib = ge * oe

    @pl.when(e == 0)
    def _init():
        acc_ref[rows, :] = contrib

    @pl.when(e > 0)
    def _accum():
        acc_ref[rows, :] = acc_ref[rows, :] + contrib

    @pl.when(e == E - 1)
    def _emit():
        out_ref[...] = acc_ref[rows, :]


@jax.jit
def _moe_dense(x2, mask2, wg_pad, fc1_w, fc1_b, fc2_w, fc2_b):
    grid = (E, T // TM)
    return pl.pallas_call(
        _moe_dense_body,
        grid=grid,
        in_specs=[
            pl.BlockSpec((TM, D), lambda e, t: (t, 0)),
            pl.BlockSpec((TM, 1), lambda e, t: (t, 0)),
            pl.BlockSpec((D, 128), lambda e, t: (0, 0)),
            pl.BlockSpec((1, D, H), lambda e, t: (e, 0, 0)),
            pl.BlockSpec((1, 1, H), lambda e, t: (e, 0, 0)),
            pl.BlockSpec((1, H, D), lambda e, t: (e, 0, 0)),
            pl.BlockSpec((1, 1, D), lambda e, t: (e, 0, 0)),
        ],
        out_specs=pl.BlockSpec((TM, D), lambda e, t: (t, 0)),
        out_shape=jax.ShapeDtypeStruct((T, D), jnp.float32),
        scratch_shapes=[
            pltpu.VMEM((T, D), jnp.float32),
            pltpu.VMEM((4, T, 1), jnp.float32),
            pltpu.VMEM((D, H), jnp.bfloat16),
            pltpu.VMEM((H, D), jnp.bfloat16),
        ],
    )(x2, mask2, wg_pad, fc1_w, fc1_b.reshape(E, 1, H),
      fc2_w, fc2_b.reshape(E, 1, D))


def kernel(x, mask, w_gate, fc1_w, fc1_b, fc2_w, fc2_b):
    x2 = x.reshape(T, D)
    mask2 = mask.reshape(T, 1)
    wg_pad = jnp.pad(w_gate, ((0, 0), (0, 128 - E)))
    y = _moe_dense(x2, mask2, wg_pad, fc1_w, fc1_b, fc2_w, fc2_b)
    return y.reshape(B, S, D)
